# Initial kernel scaffold; baseline (speedup 1.0000x reference)
import jax
import jax.numpy as jnp
from jax.experimental import pallas as pl


def kernel(x_real, x_imag, bias):
    raise NotImplementedError("write your pallas kernel here")


# speedup vs baseline: 1.9397x; 1.9397x over previous
import jax
import jax.numpy as jnp
from jax.experimental import pallas as pl


def _modrelu_block(xr_ref, xi_ref, b_ref, o_ref):
    xr = xr_ref[...]
    xi = xi_ref[...]
    b = b_ref[...]
    r2 = xr * xr + xi * xi
    inv_r = jax.lax.rsqrt(r2)
    scale = jnp.maximum(1.0 + b * inv_r, 0.0)
    scale = jnp.where(r2 > 0.0, scale, 1.0)
    o_ref[0, :, :] = scale * xr
    o_ref[1, :, :] = scale * xi


def kernel(x_real, x_imag, bias):
    n, c = x_real.shape
    bn = 512
    grid = (n // bn,)
    return pl.pallas_call(
        _modrelu_block,
        grid=grid,
        in_specs=[
            pl.BlockSpec((bn, c), lambda i: (i, 0)),
            pl.BlockSpec((bn, c), lambda i: (i, 0)),
            pl.BlockSpec((1, c), lambda i: (0, 0)),
        ],
        out_specs=pl.BlockSpec((2, bn, c), lambda i: (0, i, 0)),
        out_shape=jax.ShapeDtypeStruct((2, n, c), x_real.dtype),
    )(x_real, x_imag, bias)


# speedup vs baseline: 1.9977x; 1.0299x over previous
import jax
import jax.numpy as jnp
from jax.experimental import pallas as pl


def _modrelu_block(xr_ref, xi_ref, b_ref, o_ref):
    xr = xr_ref[...]
    xi = xi_ref[...]
    b = b_ref[...]
    r2 = xr * xr + xi * xi
    inv_r = jax.lax.rsqrt(r2)
    scale = jnp.maximum(1.0 + b * inv_r, 0.0)
    scale = jnp.where(r2 > 0.0, scale, 1.0)
    o_ref[0, :, :] = scale * xr
    o_ref[1, :, :] = scale * xi


def kernel(x_real, x_imag, bias):
    n, c = x_real.shape
    bn = 1024
    grid = (n // bn,)
    return pl.pallas_call(
        _modrelu_block,
        grid=grid,
        in_specs=[
            pl.BlockSpec((bn, c), lambda i: (i, 0)),
            pl.BlockSpec((bn, c), lambda i: (i, 0)),
            pl.BlockSpec((1, c), lambda i: (0, 0)),
        ],
        out_specs=pl.BlockSpec((2, bn, c), lambda i: (0, i, 0)),
        out_shape=jax.ShapeDtypeStruct((2, n, c), x_real.dtype),
    )(x_real, x_imag, bias)
